# Initial kernel scaffold; baseline (speedup 1.0000x reference)
import jax
import jax.numpy as jnp
from jax.experimental import pallas as pl


def kernel(x, w_dense, w_out):
    raise NotImplementedError("write your pallas kernel here")


# speedup vs baseline: 1.3082x; 1.3082x over previous
import jax
import jax.numpy as jnp
from jax.experimental import pallas as pl
from jax.experimental.pallas import tpu as pltpu


def _round_up(x, m):
    return ((x + m - 1) // m) * m


def _pick_tn(vocab, target):
    target = max(128, (target // 128) * 128)
    if vocab <= target:
        return vocab, vocab
    if vocab % 128 == 0:
        cand = target
        while cand >= 128:
            if vocab % cand == 0:
                return cand, vocab
            cand -= 128
    return target, _round_up(vocab, target)


def _fused_kernel(x_ref, wd_ref, wo_ref, o_ref, h_ref):
    @pl.when(pl.program_id(1) == 0)
    def _():
        h_ref[...] = jnp.dot(
            x_ref[...].astype(jnp.bfloat16),
            wd_ref[...].astype(jnp.bfloat16),
            preferred_element_type=jnp.float32,
        ).astype(jnp.bfloat16)

    o_ref[...] = jnp.dot(
        h_ref[...],
        wo_ref[...].astype(jnp.bfloat16),
        preferred_element_type=jnp.float32,
    ).astype(o_ref.dtype)


def kernel(x, w_dense, w_out):
    batch, seq, hidden = x.shape
    bottleneck = w_dense.shape[1]
    vocab = w_out.shape[1]
    M = batch * seq
    x2d = x.reshape(M, hidden)

    TM = min(2048, _round_up(M, 8))
    m_tiles = pl.cdiv(M, TM)
    m_pad = m_tiles * TM
    if m_pad != M:
        x2d = jnp.pad(x2d, ((0, m_pad - M), (0, 0)))

    TN, v_pad = _pick_tn(vocab, 1280)
    w_out_p = w_out if v_pad == vocab else jnp.pad(
        w_out, ((0, 0), (0, v_pad - vocab)))

    grid = (m_tiles, v_pad // TN)
    out_is = jnp.dtype(x.dtype).itemsize
    cost = pl.CostEstimate(
        flops=int(2 * M * hidden * bottleneck + 2 * M * bottleneck * vocab),
        transcendentals=0,
        bytes_accessed=int(
            m_pad * hidden * jnp.dtype(x.dtype).itemsize
            + hidden * bottleneck * jnp.dtype(w_dense.dtype).itemsize
            + m_tiles * bottleneck * v_pad * jnp.dtype(w_out.dtype).itemsize
            + m_pad * v_pad * out_is
        ),
    )

    out2d = pl.pallas_call(
        _fused_kernel,
        out_shape=jax.ShapeDtypeStruct((m_pad, v_pad), x.dtype),
        grid=grid,
        in_specs=[
            pl.BlockSpec((TM, hidden), lambda i, j: (i, 0)),
            pl.BlockSpec((hidden, bottleneck), lambda i, j: (0, 0)),
            pl.BlockSpec((bottleneck, TN), lambda i, j: (0, j)),
        ],
        out_specs=pl.BlockSpec((TM, TN), lambda i, j: (i, j)),
        scratch_shapes=[pltpu.VMEM((TM, bottleneck), jnp.bfloat16)],
        compiler_params=pltpu.CompilerParams(
            dimension_semantics=("parallel", "arbitrary"),
            vmem_limit_bytes=60 * 1024 * 1024,
        ),
        cost_estimate=cost,
    )(x2d, w_dense, w_out_p)

    out2d = out2d[:M, :vocab] if (m_pad != M or v_pad != vocab) else out2d
    return out2d.reshape(batch, seq, vocab)
